# baseline (speedup 1.0000x reference)
import jax
import jax.numpy as jnp
from jax import lax
from jax.experimental import pallas as pl
from jax.experimental.pallas import tpu as pltpu
from jax.experimental.pallas import tpu_sc as plsc

C, G, S, L, B = 4, 8192, 8, 4, 16
INFER_STEP = 2
GAMMA = 0.01
INVG = float(1.0 / GAMMA)

NC, NS = 2, 16
NW = NC * NS
BH = B // 2
NGQ = 4
GQ = G // NGQ
NG = 16
SL = S * L
CHUNK_IDX = NG * SL
NCHUNK = GQ // NG
TBL = BH * G


def _make_sc(table_has_clause_dim):
    def body(tab, idx_hbm, m_out, s_out, table_v, idx_v, pbuf, mslab, sslab,
             tsem, isem0, isem1):
        wid = lax.axis_index("s") * NC + lax.axis_index("c")
        c = wid // 8
        rem = wid - c * 8
        hb = rem // NGQ
        qg = rem - hb * NGQ
        b0 = hb * BH
        g0 = qg * GQ

        tcps = []
        for j in range(BH):
            if table_has_clause_dim:
                src = tab.at[c, b0 + j]
            else:
                src = tab.at[b0 + j]
            tcps.append(
                pltpu.async_copy(src, table_v.at[pl.ds(j * G, G)], tsem))

        isems = (isem0, isem1)

        def idx_src(k):
            kk = jnp.minimum(k, NCHUNK - 1)
            off = (c * G + g0 + kk * NG) * SL
            return idx_hbm.at[pl.ds(off, CHUNK_IDX)]

        def start_idx(k, p):
            pltpu.async_copy(idx_src(k), idx_v.at[p], isems[p])

        def wait_idx(p):
            pltpu.make_async_copy(idx_src(0), idx_v.at[p], isems[p]).wait()

        start_idx(0, 0)
        start_idx(1, 1)
        for cp in tcps:
            cp.wait()

        iota32 = lax.iota(jnp.int32, 16) * SL

        def compute(k, p):
            goff = k * NG
            for s in range(S):
                prods = [None] * BH
                for l in range(L):
                    ivec = iota32 + (s * L + l)
                    gidx = plsc.load_gather(idx_v.at[p], [ivec])
                    fidx = gidx
                    for j in range(BH):
                        if j > 0:
                            fidx = fidx + G
                        v = plsc.load_gather(table_v, [fidx])
                        if l == 0:
                            prods[j] = v
                        else:
                            prods[j] = prods[j] * v
                for j in range(BH):
                    pbuf[s, j] = prods[j]
            for j in range(BH):
                ps = [pbuf[s, j] for s in range(S)]
                m = ps[0]
                for s in range(1, S):
                    m = jnp.maximum(m, ps[s])
                acc = jnp.exp((ps[0] - m) * INVG)
                for s in range(1, S):
                    acc = acc + jnp.exp((ps[s] - m) * INVG)
                mslab[j, pl.ds(goff, NG)] = m
                sslab[j, pl.ds(goff, NG)] = acc

        def outer(i, carry):
            for u in range(2):
                k = i * 2 + u
                wait_idx(u)
                compute(k, u)
                start_idx(k + 2, u)
            return carry

        lax.fori_loop(0, NCHUNK // 2, outer, 0)
        wait_idx(0)
        wait_idx(1)

        pltpu.sync_copy(mslab, m_out.at[c, pl.ds(b0, BH), pl.ds(g0, GQ)])
        pltpu.sync_copy(sslab, s_out.at[c, pl.ds(b0, BH), pl.ds(g0, GQ)])

    tab_shape = (C, B, G) if table_has_clause_dim else (B, G)
    return pl.kernel(
        body,
        out_type=(
            jax.ShapeDtypeStruct((C, B, G), jnp.float32),
            jax.ShapeDtypeStruct((C, B, G), jnp.float32),
        ),
        mesh=plsc.VectorSubcoreMesh(
            core_axis_name="c", subcore_axis_name="s",
            num_cores=NC, num_subcores=NS,
        ),
        scratch_types=[
            pltpu.VMEM((TBL,), jnp.float32),
            pltpu.VMEM((2, CHUNK_IDX), jnp.int32),
            pltpu.VMEM((S, BH, NG), jnp.float32),
            pltpu.VMEM((BH, GQ), jnp.float32),
            pltpu.VMEM((BH, GQ), jnp.float32),
            pltpu.SemaphoreType.DMA,
            pltpu.SemaphoreType.DMA,
            pltpu.SemaphoreType.DMA,
        ],
        compiler_params=pltpu.CompilerParams(
            use_tc_tiling_on_sc=False, needs_layout_passes=False
        ),
    )


_sc_step1 = _make_sc(False)
_sc_step2 = _make_sc(True)


def _tc_body(R_ref, m_ref, s_ref, out_ref):
    t = m_ref[:] + GAMMA * jnp.log(s_ref[:])
    mx = jnp.max(t, axis=1, keepdims=True)
    r = t / jnp.maximum(mx, 1.0)
    Rc = R_ref[:]
    mm = jnp.maximum(Rc, r)
    u = mm + GAMMA * jnp.log(
        jnp.exp((Rc - mm) * INVG) + jnp.exp((r - mm) * INVG)
    )
    M = jnp.max(u)
    out_ref[:] = u / jnp.maximum(M, 1.0)


_tc_combine = pl.pallas_call(
    _tc_body,
    out_shape=jax.ShapeDtypeStruct((C, B * G), jnp.float32),
)


def kernel(x, I):
    iflat = I.reshape(C * G * SL)
    Rflat = jnp.broadcast_to(x.reshape(1, B * G), (C, B * G))
    m, acc = _sc_step1(x, iflat)
    Rflat = _tc_combine(Rflat, m.reshape(C, B * G), acc.reshape(C, B * G))
    for _ in range(INFER_STEP - 1):
        m, acc = _sc_step2(Rflat.reshape(C, B, G), iflat)
        Rflat = _tc_combine(Rflat, m.reshape(C, B * G), acc.reshape(C, B * G))
    return Rflat.reshape(C, B, G)
